# baseline (speedup 1.0000x reference)
import functools

import jax
import jax.numpy as jnp
from jax import lax
from jax.experimental import pallas as pl
from jax.experimental.pallas import tpu as pltpu
from jax.experimental.pallas import tpu_sc as plsc


@functools.lru_cache(maxsize=None)
def _make_sc_gather(V, D, CH, NBUF, G):
    info = plsc.get_sparse_core_info()
    NS = info.num_subcores
    assert V % NS == 0
    b_per_w = V // NS
    assert b_per_w % CH == 0
    n_chunks = b_per_w // CH
    assert G <= NBUF
    mesh = plsc.VectorSubcoreMesh(
        core_axis_name="c", subcore_axis_name="s", num_cores=1
    )

    @functools.partial(
        pl.kernel,
        out_type=jax.ShapeDtypeStruct((V, D), jnp.float32),
        mesh=mesh,
        scratch_types=(
            [pltpu.VMEM((b_per_w,), jnp.int32)]
            + [pltpu.VMEM((CH, D), jnp.float32) for _ in range(NBUF)]
            + [pltpu.SemaphoreType.DMA for _ in range(2 * NBUF)]
        ),
    )
    def k(table_hbm, idx_hbm, out_hbm, idx_v, *scratch):
        bufs = scratch[:NBUF]
        gsems = scratch[NBUF : 2 * NBUF]
        ssems = scratch[2 * NBUF :]
        wid = lax.axis_index("s")
        base = wid * b_per_w
        pltpu.sync_copy(idx_hbm.at[pl.ds(base, b_per_w)], idx_v)

        def gather(c):
            b = c % NBUF
            return pltpu.async_copy(
                table_hbm.at[idx_v.at[pl.ds(c * CH, CH)]], bufs[b], gsems[b]
            )

        g = {}
        s = {}
        for c in range(min(G, n_chunks)):
            g[c] = gather(c)
        for c in range(n_chunks):
            b = c % NBUF
            g[c].wait()
            s[c] = pltpu.async_copy(
                bufs[b], out_hbm.at[pl.ds(base + c * CH, CH)], ssems[b]
            )
            nxt = c + G
            if nxt < n_chunks:
                old = nxt - NBUF
                if old >= 0:
                    s[old].wait()
                g[nxt] = gather(nxt)
        for c in range(max(0, n_chunks - NBUF), n_chunks):
            s[c].wait()

    return k


def kernel(table, length):
    V, D = table.shape
    idx = jnp.minimum(
        jnp.arange(V, dtype=jnp.int32), jnp.asarray(length, jnp.int32) - 1
    )
    return _make_sc_gather(V, D, 32, 3, 2)(table, idx)


# speedup vs baseline: 1.1156x; 1.1156x over previous
import functools

import jax
import jax.numpy as jnp
from jax import lax
from jax.experimental import pallas as pl
from jax.experimental.pallas import tpu as pltpu
from jax.experimental.pallas import tpu_sc as plsc


@functools.lru_cache(maxsize=None)
def _make_sc_gather(V, D, CH, NBUF, G):
    info = plsc.get_sparse_core_info()
    NS = info.num_subcores
    assert V % (NS * CH) == 0
    n_chunks = V // (NS * CH)
    assert G <= NBUF
    mesh = plsc.VectorSubcoreMesh(
        core_axis_name="c", subcore_axis_name="s", num_cores=1
    )

    @functools.partial(
        pl.kernel,
        out_type=jax.ShapeDtypeStruct((V, D), jnp.float32),
        mesh=mesh,
        scratch_types=(
            [pltpu.VMEM((CH,), jnp.int32) for _ in range(NBUF)]
            + [pltpu.VMEM((CH, D), jnp.float32) for _ in range(NBUF)]
            + [pltpu.SemaphoreType.DMA for _ in range(2 * NBUF)]
        ),
    )
    def k(table_hbm, idx_hbm, out_hbm, *scratch):
        ixv = scratch[:NBUF]
        bufs = scratch[NBUF : 2 * NBUF]
        gsems = scratch[2 * NBUF : 3 * NBUF]
        ssems = scratch[3 * NBUF :]
        wid = lax.axis_index("s")

        def block(c):
            return c * NS + wid

        def gather(c):
            b = c % NBUF
            pltpu.sync_copy(idx_hbm.at[pl.ds(block(c) * CH, CH)], ixv[b])
            return pltpu.async_copy(
                table_hbm.at[ixv[b]], bufs[b], gsems[b]
            )

        g = {}
        s = {}
        for c in range(min(G, n_chunks)):
            g[c] = gather(c)
        for c in range(n_chunks):
            b = c % NBUF
            g[c].wait()
            s[c] = pltpu.async_copy(
                bufs[b], out_hbm.at[pl.ds(block(c) * CH, CH)], ssems[b]
            )
            nxt = c + G
            if nxt < n_chunks:
                old = nxt - NBUF
                if old >= 0:
                    s[old].wait()
                g[nxt] = gather(nxt)
        for c in range(max(0, n_chunks - NBUF), n_chunks):
            s[c].wait()

    return k


def kernel(table, length):
    V, D = table.shape
    idx = jnp.minimum(
        jnp.arange(V, dtype=jnp.int32), jnp.asarray(length, jnp.int32) - 1
    )
    return _make_sc_gather(V, D, 16, 7, 4)(table, idx)


# speedup vs baseline: 1.2044x; 1.0796x over previous
import functools

import jax
import jax.numpy as jnp
from jax import lax
from jax.experimental import pallas as pl
from jax.experimental.pallas import tpu as pltpu
from jax.experimental.pallas import tpu_sc as plsc


@functools.lru_cache(maxsize=None)
def _make_sc_gather(V, D, CH, NBUF, G):
    info = plsc.get_sparse_core_info()
    NW = info.num_cores * info.num_subcores
    assert V % NW == 0
    b_per_w = V // NW
    assert b_per_w % CH == 0 and b_per_w % 8 == 0
    n_chunks = b_per_w // CH
    assert G <= NBUF
    mesh = plsc.VectorSubcoreMesh(core_axis_name="c", subcore_axis_name="s")

    @functools.partial(
        pl.kernel,
        out_type=jax.ShapeDtypeStruct((V, D), jnp.float32),
        mesh=mesh,
        scratch_types=(
            [pltpu.VMEM((b_per_w,), jnp.int32)]
            + [pltpu.VMEM((CH, D), jnp.float32) for _ in range(NBUF)]
            + [pltpu.SemaphoreType.DMA for _ in range(2 * NBUF)]
        ),
    )
    def k(table_hbm, idx_hbm, out_hbm, idx_v, *scratch):
        bufs = scratch[:NBUF]
        gsems = scratch[NBUF : 2 * NBUF]
        ssems = scratch[2 * NBUF :]
        wid = lax.axis_index("s") * info.num_cores + lax.axis_index("c")
        base = wid * b_per_w
        pltpu.sync_copy(idx_hbm.at[pl.ds(base, b_per_w)], idx_v)

        def gather(c):
            b = c % NBUF
            return pltpu.async_copy(
                table_hbm.at[idx_v.at[pl.ds(c * CH, CH)]], bufs[b], gsems[b]
            )

        g = {}
        s = {}
        for c in range(min(G, n_chunks)):
            g[c] = gather(c)
        for c in range(n_chunks):
            b = c % NBUF
            g[c].wait()
            s[c] = pltpu.async_copy(
                bufs[b], out_hbm.at[pl.ds(base + c * CH, CH)], ssems[b]
            )
            nxt = c + G
            if nxt < n_chunks:
                old = nxt - NBUF
                if old >= 0:
                    s[old].wait()
                g[nxt] = gather(nxt)
        for c in range(max(0, n_chunks - NBUF), n_chunks):
            s[c].wait()

    return k


def kernel(table, length):
    V, D = table.shape
    idx = jnp.minimum(
        jnp.arange(V, dtype=jnp.int32), jnp.asarray(length, jnp.int32) - 1
    )
    return _make_sc_gather(V, D, 32, 3, 2)(table, idx)


# speedup vs baseline: 1.2512x; 1.0389x over previous
import functools

import jax
import jax.numpy as jnp
from jax import lax
from jax.experimental import pallas as pl
from jax.experimental.pallas import tpu as pltpu
from jax.experimental.pallas import tpu_sc as plsc


@functools.lru_cache(maxsize=None)
def _make_sc_gather(V, D, CH, NBUF, G):
    info = plsc.get_sparse_core_info()
    NW = info.num_cores * info.num_subcores
    assert V % NW == 0
    b_per_w = V // NW
    assert b_per_w % CH == 0 and b_per_w % 8 == 0
    n_chunks = b_per_w // CH
    assert G <= NBUF
    mesh = plsc.VectorSubcoreMesh(core_axis_name="c", subcore_axis_name="s")

    @functools.partial(
        pl.kernel,
        out_type=jax.ShapeDtypeStruct((V, D), jnp.float32),
        mesh=mesh,
        scratch_types=(
            [pltpu.VMEM((b_per_w,), jnp.int32)]
            + [pltpu.VMEM((CH, D), jnp.float32) for _ in range(NBUF)]
            + [pltpu.SemaphoreType.DMA for _ in range(2 * NBUF)]
        ),
    )
    def k(table_hbm, idx_hbm, out_hbm, idx_v, *scratch):
        bufs = scratch[:NBUF]
        gsems = scratch[NBUF : 2 * NBUF]
        ssems = scratch[2 * NBUF :]
        wid = lax.axis_index("s") * info.num_cores + lax.axis_index("c")
        base = wid * b_per_w
        pltpu.sync_copy(idx_hbm.at[pl.ds(base, b_per_w)], idx_v)

        def gather(c):
            b = c % NBUF
            return pltpu.async_copy(
                table_hbm.at[idx_v.at[pl.ds(c * CH, CH)]], bufs[b], gsems[b]
            )

        g = {}
        s = {}
        for c in range(min(G, n_chunks)):
            g[c] = gather(c)
        for c in range(n_chunks):
            b = c % NBUF
            g[c].wait()
            s[c] = pltpu.async_copy(
                bufs[b], out_hbm.at[pl.ds(base + c * CH, CH)], ssems[b]
            )
            nxt = c + G
            if nxt < n_chunks:
                old = nxt - NBUF
                if old >= 0:
                    s[old].wait()
                g[nxt] = gather(nxt)
        for c in range(max(0, n_chunks - NBUF), n_chunks):
            s[c].wait()

    return k


def kernel(table, length):
    V, D = table.shape
    idx = jnp.minimum(
        jnp.arange(V, dtype=jnp.int32), jnp.asarray(length, jnp.int32) - 1
    )
    return _make_sc_gather(V, D, 16, 7, 4)(table, idx)


# speedup vs baseline: 1.2557x; 1.0036x over previous
import functools

import jax
import jax.numpy as jnp
from jax import lax
from jax.experimental import pallas as pl
from jax.experimental.pallas import tpu as pltpu
from jax.experimental.pallas import tpu_sc as plsc


@functools.lru_cache(maxsize=None)
def _make_sc_gather(V, D, CH, NBUF, G):
    info = plsc.get_sparse_core_info()
    NW = info.num_cores * info.num_subcores
    assert V % NW == 0
    b_per_w = V // NW
    assert b_per_w % CH == 0 and b_per_w % 8 == 0
    n_chunks = b_per_w // CH
    assert G <= NBUF
    mesh = plsc.VectorSubcoreMesh(core_axis_name="c", subcore_axis_name="s")

    @functools.partial(
        pl.kernel,
        out_type=jax.ShapeDtypeStruct((V, D), jnp.float32),
        mesh=mesh,
        scratch_types=(
            [pltpu.VMEM((b_per_w,), jnp.int32)]
            + [pltpu.VMEM((CH, D), jnp.float32) for _ in range(NBUF)]
            + [pltpu.SemaphoreType.DMA for _ in range(2 * NBUF)]
        ),
    )
    def k(table_hbm, idx_hbm, out_hbm, idx_v, *scratch):
        bufs = scratch[:NBUF]
        gsems = scratch[NBUF : 2 * NBUF]
        ssems = scratch[2 * NBUF :]
        wid = lax.axis_index("s") * info.num_cores + lax.axis_index("c")
        base = wid * b_per_w
        pltpu.sync_copy(idx_hbm.at[pl.ds(base, b_per_w)], idx_v)

        def gather(c):
            b = c % NBUF
            return pltpu.async_copy(
                table_hbm.at[idx_v.at[pl.ds(c * CH, CH)]], bufs[b], gsems[b]
            )

        g = {}
        s = {}
        for c in range(min(G, n_chunks)):
            g[c] = gather(c)
        for c in range(n_chunks):
            b = c % NBUF
            g[c].wait()
            s[c] = pltpu.async_copy(
                bufs[b], out_hbm.at[pl.ds(base + c * CH, CH)], ssems[b]
            )
            nxt = c + G
            if nxt < n_chunks:
                old = nxt - NBUF
                if old >= 0:
                    s[old].wait()
                g[nxt] = gather(nxt)
        for c in range(max(0, n_chunks - NBUF), n_chunks):
            s[c].wait()

    return k


def kernel(table, length):
    V, D = table.shape
    idx = jnp.minimum(
        jnp.arange(V, dtype=jnp.int32), jnp.asarray(length, jnp.int32) - 1
    )
    return _make_sc_gather(V, D, 16, 7, 6)(table, idx)
